# baseline (speedup 1.0000x reference)
import functools

import jax
import jax.numpy as jnp
from jax import lax
from jax.experimental import pallas as pl
from jax.experimental.pallas import tpu as pltpu
from jax.experimental.pallas import tpu_sc as plsc

N = 50000
E = 800000
ND = 64
HN = ND // 2
ED = 16
G = 256
L = 3

NC = 2
NS = 16

EPC = 128
NCH = E // EPC
CPS = (NCH + NS - 1) // NS
NP = 50048
RPS = NP // NS
ZR = 136
NZC = RPS // ZR

_LP = (3.50755203726294e-06, 0.9997924566268921, -0.49697792530059814,
       0.31459054350852966, -0.1887826770544052, 0.0817268118262291,
       -0.01720806024968624)

_sc_mesh = plsc.VectorSubcoreMesh(core_axis_name="c", subcore_axis_name="s")


def _gate(gf, gs):
    sig = 1.0 / (1.0 + jnp.exp(-gf))
    y = jnp.exp(-jnp.abs(gs))
    lp = _LP[6]
    for k in (5, 4, 3, 2, 1, 0):
        lp = lp * y + _LP[k]
    sp = jnp.maximum(gs, 0.0) + lp
    return sig * sp


@functools.partial(
    pl.kernel,
    out_type=jax.ShapeDtypeStruct((NC, NP, HN), jnp.float32),
    mesh=_sc_mesh,
    scratch_types=[
        pltpu.VMEM((EPC,), jnp.int32),
        pltpu.VMEM((EPC,), jnp.int32),
        pltpu.VMEM((EPC, ND), jnp.float32),
        pltpu.VMEM((EPC, ND), jnp.float32),
        pltpu.VMEM((EPC, ND), jnp.float32),
        pltpu.VMEM((EPC, HN), jnp.float32),
        pltpu.VMEM_SHARED((NP, HN), jnp.float32),
        pltpu.SemaphoreType.DMA,
        pltpu.SemaphoreType.DMA,
        pltpu.SemaphoreType.DMA,
    ],
    compiler_params=pltpu.CompilerParams(use_tc_tiling_on_sc=False),
)
def _sc_layer(p0, q0, c0, p1, q1, c1, dst, src, agg_hbm,
              di, si, pr, qr, cr, mr, agg_sh, s1, s2, s3):
    cid = lax.axis_index("c")
    sid = lax.axis_index("s")

    zeros = jnp.zeros((16,), jnp.float32)

    def zrow(i, carry):
        mr[i, pl.ds(0, 16)] = zeros
        mr[i, pl.ds(16, 16)] = zeros
        return carry

    lax.fori_loop(0, EPC, zrow, 0)

    zfull = RPS // EPC
    zrem = RPS - zfull * EPC

    def zblk(z, carry):
        pltpu.sync_copy(mr, agg_sh.at[pl.ds(sid * RPS + z * EPC, EPC)])
        return carry

    lax.fori_loop(0, zfull, zblk, 0)
    pltpu.sync_copy(mr.at[pl.ds(0, zrem)],
                    agg_sh.at[pl.ds(sid * RPS + zfull * EPC, zrem)])
    plsc.subcore_barrier()

    def chunk(i, carry):
        ci = sid + NS * i

        @pl.when(ci < NCH)
        def _():
            base = ci * EPC
            pltpu.sync_copy(dst.at[pl.ds(base, EPC)], di)
            pltpu.sync_copy(src.at[pl.ds(base, EPC)], si)

            @pl.when(cid == 0)
            def _():
                cp = pltpu.async_copy(p0.at[di], pr, s1)
                cq = pltpu.async_copy(q0.at[si], qr, s2)
                cc = pltpu.async_copy(c0.at[pl.ds(base, EPC)], cr, s3)
                cp.wait()
                cq.wait()
                cc.wait()

            @pl.when(cid == 1)
            def _():
                cp = pltpu.async_copy(p1.at[di], pr, s1)
                cq = pltpu.async_copy(q1.at[si], qr, s2)
                cc = pltpu.async_copy(c1.at[pl.ds(base, EPC)], cr, s3)
                cp.wait()
                cq.wait()
                cc.wait()

            @plsc.parallel_loop(0, EPC, unroll=4)
            def row(r):
                for g in range(HN // 16):
                    fsl = pl.ds(16 * g, 16)
                    ssl = pl.ds(HN + 16 * g, 16)
                    gf = pr[r, fsl] + qr[r, fsl] + cr[r, fsl]
                    gs = pr[r, ssl] + qr[r, ssl] + cr[r, ssl]
                    mr[r, fsl] = _gate(gf, gs)

            pltpu.sync_copy(mr, agg_sh.at[di], add=True)

        return carry

    lax.fori_loop(0, CPS, chunk, 0)
    plsc.subcore_barrier()

    def wblk(z, carry):
        r0 = sid * RPS + z * ZR
        pltpu.sync_copy(agg_sh.at[pl.ds(r0, ZR)], agg_hbm.at[cid, pl.ds(r0, ZR)])
        return carry

    lax.fori_loop(0, NZC, wblk, 0)


def kernel(x, edge_index, edge_attr, batch, W_emb, b_emb, Wf, bf, Ws, bs, gamma, beta, W_fc, b_fc, W_out, b_out):
    src = edge_index[0]
    dst = edge_index[1]
    h = x @ W_emb + b_emb
    for l in range(L):
        Wi = jnp.concatenate([Wf[l, :ND], Ws[l, :ND]], axis=1)
        Wj = jnp.concatenate([Wf[l, ND:2 * ND], Ws[l, ND:2 * ND]], axis=1)
        We = jnp.concatenate([Wf[l, 2 * ND:], Ws[l, 2 * ND:]], axis=1)
        bb = jnp.concatenate([bf[l], bs[l]])[None, :]
        cols0 = jnp.concatenate([jnp.arange(HN), ND + jnp.arange(HN)])
        cols1 = cols0 + HN
        P = h @ Wi
        Q = h @ Wj
        C = edge_attr @ We + bb
        p0, p1 = P[:, cols0], P[:, cols1]
        q0, q1 = Q[:, cols0], Q[:, cols1]
        c0, c1 = C[:, cols0], C[:, cols1]
        aggh = _sc_layer(p0, q0, c0, p1, q1, c1, dst, src)
        agg = jnp.concatenate([aggh[0, :N], aggh[1, :N]], axis=1)
        mean = jnp.mean(agg, axis=0)
        var = jnp.var(agg, axis=0)
        agg = (agg - mean) / jnp.sqrt(var + 1e-5) * gamma[l] + beta[l]
        h = h + agg
    sums = jax.ops.segment_sum(h, batch, num_segments=G)
    counts = jax.ops.segment_sum(jnp.ones((N, 1), dtype=h.dtype), batch, num_segments=G)
    pooled = sums / jnp.maximum(counts, 1.0)
    y = jax.nn.softplus(pooled)
    y = y @ W_fc + b_fc
    y = jax.nn.softplus(y)
    y = y @ W_out + b_out
    return y


# speedup vs baseline: 1.1509x; 1.1509x over previous
import functools

import jax
import jax.numpy as jnp
from jax import lax
from jax.experimental import pallas as pl
from jax.experimental.pallas import tpu as pltpu
from jax.experimental.pallas import tpu_sc as plsc

N = 50000
E = 800000
ND = 64
HN = ND // 2
ED = 16
G = 256
L = 3

NC = 2
NS = 16

EPC = 40
NCH = E // EPC
CPS = NCH // NS
NP = 50048
RPS = NP // NS
ZR = 136
NZC = RPS // ZR

_LP = (3.50755203726294e-06, 0.9997924566268921, -0.49697792530059814,
       0.31459054350852966, -0.1887826770544052, 0.0817268118262291,
       -0.01720806024968624)

_sc_mesh = plsc.VectorSubcoreMesh(core_axis_name="c", subcore_axis_name="s")


def _gate(gf, gs):
    sig = 1.0 / (1.0 + jnp.exp(-gf))
    y = jnp.exp(-jnp.abs(gs))
    lp = _LP[6]
    for k in (5, 4, 3, 2, 1, 0):
        lp = lp * y + _LP[k]
    sp = jnp.maximum(gs, 0.0) + lp
    return sig * sp


@functools.partial(
    pl.kernel,
    out_type=jax.ShapeDtypeStruct((NC, NP, HN), jnp.float32),
    mesh=_sc_mesh,
    scratch_types=[
        pltpu.VMEM((2, EPC), jnp.int32),
        pltpu.VMEM((2, EPC), jnp.int32),
        pltpu.VMEM((2, EPC), jnp.int32),
        pltpu.VMEM((EPC, ND), jnp.float32),
        pltpu.VMEM((EPC, ND), jnp.float32),
        pltpu.VMEM((EPC, ND), jnp.float32),
        pltpu.VMEM((EPC, ND), jnp.float32),
        pltpu.VMEM((EPC, ND), jnp.float32),
        pltpu.VMEM((EPC, ND), jnp.float32),
        pltpu.VMEM((EPC, HN), jnp.float32),
        pltpu.VMEM_SHARED((NP, HN), jnp.float32),
        pltpu.SemaphoreType.DMA,
        pltpu.SemaphoreType.DMA,
        pltpu.SemaphoreType.DMA,
        pltpu.SemaphoreType.DMA,
        pltpu.SemaphoreType.DMA,
        pltpu.SemaphoreType.DMA,
    ],
    compiler_params=pltpu.CompilerParams(use_tc_tiling_on_sc=False),
)
def _sc_layer(p0, q0, c0, p1, q1, c1, dst2, src2, agg_hbm,
              di, si, dsc, pr0, pr1, qr0, qr1, cr0, cr1, mr, agg_sh,
              ig0, ig1, is0, is1, g0, g1):
    cid = lax.axis_index("c")
    sid = lax.axis_index("s")
    prs = (pr0, pr1)
    qrs = (qr0, qr1)
    crs = (cr0, cr1)
    igs = (ig0, ig1)
    iss = (is0, is1)
    gs_ = (g0, g1)
    tb = sid * CPS

    zeros = jnp.zeros((16,), jnp.float32)

    def zrow(i, carry):
        mr[i, pl.ds(0, 16)] = zeros
        mr[i, pl.ds(16, 16)] = zeros
        return carry

    lax.fori_loop(0, EPC, zrow, 0)

    zfull = RPS // EPC
    zrem = RPS - zfull * EPC

    def zblk(z, carry):
        pltpu.sync_copy(mr, agg_sh.at[pl.ds(sid * RPS + z * EPC, EPC)])
        return carry

    lax.fori_loop(0, zfull, zblk, 0)
    if zrem:
        pltpu.sync_copy(mr.at[pl.ds(0, zrem)],
                        agg_sh.at[pl.ds(sid * RPS + zfull * EPC, zrem)])
    plsc.subcore_barrier()

    def issue_idx(k, p):
        pltpu.async_copy(dst2.at[tb + k], di.at[p], igs[p])
        pltpu.async_copy(src2.at[tb + k], si.at[p], igs[p])

    def issue_dsc(k, p):
        pltpu.async_copy(dst2.at[tb + k], dsc.at[p], iss[p])

    def wait_idx(p):
        pltpu.make_async_copy(dst2.at[0], di.at[p], igs[p]).wait()
        pltpu.make_async_copy(dst2.at[0], si.at[p], igs[p]).wait()

    def wait_dsc(p):
        pltpu.make_async_copy(dst2.at[0], dsc.at[p], iss[p]).wait()

    def issue_gathers(k, p):
        @pl.when(cid == 0)
        def _():
            pltpu.async_copy(p0.at[di.at[p]], prs[p], gs_[p])
            pltpu.async_copy(q0.at[si.at[p]], qrs[p], gs_[p])
            pltpu.async_copy(c0.at[pl.ds((tb + k) * EPC, EPC)], crs[p], gs_[p])

        @pl.when(cid == 1)
        def _():
            pltpu.async_copy(p1.at[di.at[p]], prs[p], gs_[p])
            pltpu.async_copy(q1.at[si.at[p]], qrs[p], gs_[p])
            pltpu.async_copy(c1.at[pl.ds((tb + k) * EPC, EPC)], crs[p], gs_[p])

    def wait_gathers(p):
        pltpu.make_async_copy(c0.at[pl.ds(0, EPC)], prs[p], gs_[p]).wait()
        pltpu.make_async_copy(c0.at[pl.ds(0, EPC)], qrs[p], gs_[p]).wait()
        pltpu.make_async_copy(c0.at[pl.ds(0, EPC)], crs[p], gs_[p]).wait()

    issue_idx(0, 0)
    issue_dsc(0, 0)
    issue_idx(1, 1)
    issue_dsc(1, 1)
    wait_idx(0)
    issue_gathers(0, 0)

    def pair(t, carry):
        for p in (0, 1):
            k = 2 * t + p

            @pl.when(k + 1 < CPS)
            def _():
                wait_idx(1 - p)
                issue_gathers(k + 1, 1 - p)

            wait_gathers(p)

            @pl.when(k + 2 < CPS)
            def _():
                issue_idx(k + 2, p)

            pr_, qr_, cr_ = prs[p], qrs[p], crs[p]

            @plsc.parallel_loop(0, EPC, unroll=4)
            def row(r):
                for g in range(HN // 16):
                    fsl = pl.ds(16 * g, 16)
                    ssl = pl.ds(HN + 16 * g, 16)
                    gf = pr_[r, fsl] + qr_[r, fsl] + cr_[r, fsl]
                    gg = pr_[r, ssl] + qr_[r, ssl] + cr_[r, ssl]
                    mr[r, fsl] = _gate(gf, gg)

            wait_dsc(p)
            pltpu.sync_copy(mr, agg_sh.at[dsc.at[p]], add=True)

            @pl.when(k + 2 < CPS)
            def _():
                issue_dsc(k + 2, p)

        return carry

    lax.fori_loop(0, CPS // 2, pair, 0)
    plsc.subcore_barrier()

    def wblk(z, carry):
        r0 = sid * RPS + z * ZR
        pltpu.sync_copy(agg_sh.at[pl.ds(r0, ZR)], agg_hbm.at[cid, pl.ds(r0, ZR)])
        return carry

    lax.fori_loop(0, NZC, wblk, 0)


def kernel(x, edge_index, edge_attr, batch, W_emb, b_emb, Wf, bf, Ws, bs, gamma, beta, W_fc, b_fc, W_out, b_out):
    src = edge_index[0]
    dst = edge_index[1]
    dst2 = dst.reshape(NCH, EPC)
    src2 = src.reshape(NCH, EPC)
    h = x @ W_emb + b_emb
    for l in range(L):
        Wi = jnp.concatenate([Wf[l, :ND], Ws[l, :ND]], axis=1)
        Wj = jnp.concatenate([Wf[l, ND:2 * ND], Ws[l, ND:2 * ND]], axis=1)
        We = jnp.concatenate([Wf[l, 2 * ND:], Ws[l, 2 * ND:]], axis=1)
        bb = jnp.concatenate([bf[l], bs[l]])[None, :]
        cols0 = jnp.concatenate([jnp.arange(HN), ND + jnp.arange(HN)])
        cols1 = cols0 + HN
        P = h @ Wi
        Q = h @ Wj
        C = edge_attr @ We + bb
        p0, p1 = P[:, cols0], P[:, cols1]
        q0, q1 = Q[:, cols0], Q[:, cols1]
        c0, c1 = C[:, cols0], C[:, cols1]
        aggh = _sc_layer(p0, q0, c0, p1, q1, c1, dst2, src2)
        agg = jnp.concatenate([aggh[0, :N], aggh[1, :N]], axis=1)
        mean = jnp.mean(agg, axis=0)
        var = jnp.var(agg, axis=0)
        agg = (agg - mean) / jnp.sqrt(var + 1e-5) * gamma[l] + beta[l]
        h = h + agg
    sums = jax.ops.segment_sum(h, batch, num_segments=G)
    counts = jax.ops.segment_sum(jnp.ones((N, 1), dtype=h.dtype), batch, num_segments=G)
    pooled = sums / jnp.maximum(counts, 1.0)
    y = jax.nn.softplus(pooled)
    y = y @ W_fc + b_fc
    y = jax.nn.softplus(y)
    y = y @ W_out + b_out
    return y


# speedup vs baseline: 2.1449x; 1.8636x over previous
import functools

import jax
import jax.numpy as jnp
from jax import lax
from jax.experimental import pallas as pl
from jax.experimental.pallas import tpu as pltpu
from jax.experimental.pallas import tpu_sc as plsc

N = 50000
E = 800000
ND = 64
HN = ND // 2
ED = 16
G = 256
L = 3

NC = 2
NS = 16

EPC = 40
NCH = E // EPC
CPS = NCH // NS
NP = 50048
RPS = NP // NS
ZR = 136
NZC = RPS // ZR

_LP = (3.50755203726294e-06, 0.9997924566268921, -0.49697792530059814,
       0.31459054350852966, -0.1887826770544052, 0.0817268118262291,
       -0.01720806024968624)

_sc_mesh = plsc.VectorSubcoreMesh(core_axis_name="c", subcore_axis_name="s")


def _gate(gf, gs):
    sig = 1.0 / (1.0 + jnp.exp(-gf))
    y = jnp.exp(-jnp.abs(gs))
    lp = _LP[6]
    for k in (5, 4, 3, 2, 1, 0):
        lp = lp * y + _LP[k]
    sp = jnp.maximum(gs, 0.0) + lp
    return sig * sp


@functools.partial(
    pl.kernel,
    out_type=jax.ShapeDtypeStruct((NC, NP, HN), jnp.float32),
    mesh=_sc_mesh,
    scratch_types=[
        pltpu.VMEM((2, EPC), jnp.int32),
        pltpu.VMEM((2, EPC), jnp.int32),
        pltpu.VMEM((2, EPC), jnp.int32),
        pltpu.VMEM((EPC, ND), jnp.float32),
        pltpu.VMEM((EPC, ND), jnp.float32),
        pltpu.VMEM((EPC, ND), jnp.float32),
        pltpu.VMEM((EPC, ND), jnp.float32),
        pltpu.VMEM((EPC, ND), jnp.float32),
        pltpu.VMEM((EPC, ND), jnp.float32),
        pltpu.VMEM((EPC, HN), jnp.float32),
        pltpu.VMEM_SHARED((NP, HN), jnp.float32),
        pltpu.SemaphoreType.DMA,
        pltpu.SemaphoreType.DMA,
        pltpu.SemaphoreType.DMA,
        pltpu.SemaphoreType.DMA,
        pltpu.SemaphoreType.DMA,
        pltpu.SemaphoreType.DMA,
    ],
    compiler_params=pltpu.CompilerParams(use_tc_tiling_on_sc=False),
)
def _sc_layer(p0, q0, c0, p1, q1, c1, dst2, src2, agg_hbm,
              di, si, dsc, pr0, pr1, qr0, qr1, cr0, cr1, mr, agg_sh,
              ig0, ig1, is0, is1, g0, g1):
    cid = lax.axis_index("c")
    sid = lax.axis_index("s")
    prs = (pr0, pr1)
    qrs = (qr0, qr1)
    crs = (cr0, cr1)
    igs = (ig0, ig1)
    iss = (is0, is1)
    gs_ = (g0, g1)
    tb = sid * CPS

    zeros = jnp.zeros((16,), jnp.float32)

    def zrow(i, carry):
        mr[i, pl.ds(0, 16)] = zeros
        mr[i, pl.ds(16, 16)] = zeros
        return carry

    lax.fori_loop(0, EPC, zrow, 0)

    zfull = RPS // EPC
    zrem = RPS - zfull * EPC

    def zblk(z, carry):
        pltpu.sync_copy(mr, agg_sh.at[pl.ds(sid * RPS + z * EPC, EPC)])
        return carry

    lax.fori_loop(0, zfull, zblk, 0)
    if zrem:
        pltpu.sync_copy(mr.at[pl.ds(0, zrem)],
                        agg_sh.at[pl.ds(sid * RPS + zfull * EPC, zrem)])
    plsc.subcore_barrier()

    def issue_idx(k, p):
        pltpu.async_copy(dst2.at[tb + k], di.at[p], igs[p])
        pltpu.async_copy(src2.at[tb + k], si.at[p], igs[p])

    def issue_dsc(k, p):
        pltpu.async_copy(dst2.at[tb + k], dsc.at[p], iss[p])

    def wait_idx(p):
        pltpu.make_async_copy(dst2.at[0], di.at[p], igs[p]).wait()
        pltpu.make_async_copy(dst2.at[0], si.at[p], igs[p]).wait()

    def wait_dsc(p):
        pltpu.make_async_copy(dst2.at[0], dsc.at[p], iss[p]).wait()

    def issue_gathers(k, p):
        @pl.when(cid == 0)
        def _():
            pltpu.async_copy(p0.at[di.at[p]], prs[p], gs_[p])
            pltpu.async_copy(q0.at[si.at[p]], qrs[p], gs_[p])
            pltpu.async_copy(c0.at[pl.ds((tb + k) * EPC, EPC)], crs[p], gs_[p])

        @pl.when(cid == 1)
        def _():
            pltpu.async_copy(p1.at[di.at[p]], prs[p], gs_[p])
            pltpu.async_copy(q1.at[si.at[p]], qrs[p], gs_[p])
            pltpu.async_copy(c1.at[pl.ds((tb + k) * EPC, EPC)], crs[p], gs_[p])

    def wait_gathers(p):
        pltpu.make_async_copy(c0.at[pl.ds(0, EPC)], prs[p], gs_[p]).wait()
        pltpu.make_async_copy(c0.at[pl.ds(0, EPC)], qrs[p], gs_[p]).wait()
        pltpu.make_async_copy(c0.at[pl.ds(0, EPC)], crs[p], gs_[p]).wait()

    issue_idx(0, 0)
    issue_dsc(0, 0)
    issue_idx(1, 1)
    issue_dsc(1, 1)
    wait_idx(0)
    issue_gathers(0, 0)

    def pair(t, carry):
        for p in (0, 1):
            k = 2 * t + p

            @pl.when(k + 1 < CPS)
            def _():
                wait_idx(1 - p)
                issue_gathers(k + 1, 1 - p)

            wait_gathers(p)

            @pl.when(k + 2 < CPS)
            def _():
                issue_idx(k + 2, p)

            pr_, qr_, cr_ = prs[p], qrs[p], crs[p]

            @plsc.parallel_loop(0, EPC, unroll=4)
            def row(r):
                for g in range(HN // 16):
                    fsl = pl.ds(16 * g, 16)
                    ssl = pl.ds(HN + 16 * g, 16)
                    gf = pr_[r, fsl] + qr_[r, fsl] + cr_[r, fsl]
                    gg = pr_[r, ssl] + qr_[r, ssl] + cr_[r, ssl]
                    mr[r, fsl] = _gate(gf, gg)

            wait_dsc(p)
            pltpu.sync_copy(mr, agg_sh.at[dsc.at[p]], add=True)

            @pl.when(k + 2 < CPS)
            def _():
                issue_dsc(k + 2, p)

        return carry

    lax.fori_loop(0, CPS // 2, pair, 0)
    plsc.subcore_barrier()

    def wblk(z, carry):
        r0 = sid * RPS + z * ZR
        pltpu.sync_copy(agg_sh.at[pl.ds(r0, ZR)], agg_hbm.at[cid, pl.ds(r0, ZR)])
        return carry

    lax.fori_loop(0, NZC, wblk, 0)


def kernel(x, edge_index, edge_attr, batch, W_emb, b_emb, Wf, bf, Ws, bs, gamma, beta, W_fc, b_fc, W_out, b_out):
    src = edge_index[0]
    dst = edge_index[1]
    dst2 = dst.reshape(NCH, EPC)
    src2 = src.reshape(NCH, EPC)
    h = x @ W_emb + b_emb
    for l in range(L):
        Wi0 = jnp.concatenate([Wf[l, :ND, :HN], Ws[l, :ND, :HN]], axis=1)
        Wi1 = jnp.concatenate([Wf[l, :ND, HN:], Ws[l, :ND, HN:]], axis=1)
        Wj0 = jnp.concatenate([Wf[l, ND:2 * ND, :HN], Ws[l, ND:2 * ND, :HN]], axis=1)
        Wj1 = jnp.concatenate([Wf[l, ND:2 * ND, HN:], Ws[l, ND:2 * ND, HN:]], axis=1)
        We0 = jnp.concatenate([Wf[l, 2 * ND:, :HN], Ws[l, 2 * ND:, :HN]], axis=1)
        We1 = jnp.concatenate([Wf[l, 2 * ND:, HN:], Ws[l, 2 * ND:, HN:]], axis=1)
        bb0 = jnp.concatenate([bf[l, :HN], bs[l, :HN]])[None, :]
        bb1 = jnp.concatenate([bf[l, HN:], bs[l, HN:]])[None, :]
        p0, p1 = h @ Wi0, h @ Wi1
        q0, q1 = h @ Wj0, h @ Wj1
        c0 = edge_attr @ We0 + bb0
        c1 = edge_attr @ We1 + bb1
        aggh = _sc_layer(p0, q0, c0, p1, q1, c1, dst2, src2)
        agg = jnp.concatenate([aggh[0, :N], aggh[1, :N]], axis=1)
        mean = jnp.mean(agg, axis=0)
        var = jnp.var(agg, axis=0)
        agg = (agg - mean) / jnp.sqrt(var + 1e-5) * gamma[l] + beta[l]
        h = h + agg
    sums = jax.ops.segment_sum(h, batch, num_segments=G)
    counts = jax.ops.segment_sum(jnp.ones((N, 1), dtype=h.dtype), batch, num_segments=G)
    pooled = sums / jnp.maximum(counts, 1.0)
    y = jax.nn.softplus(pooled)
    y = y @ W_fc + b_fc
    y = jax.nn.softplus(y)
    y = y @ W_out + b_out
    return y
